# baseline (speedup 1.0000x reference)
import functools

import jax
import jax.numpy as jnp
from jax import lax
from jax.experimental import pallas as pl
from jax.experimental.pallas import tpu as pltpu
from jax.experimental.pallas import tpu_sc as plsc

_SUB = 8
_RPW = 64


def _diag_gather_sc(xt, batch, dim, dpad):
    mesh = plsc.VectorSubcoreMesh(core_axis_name="c", subcore_axis_name="s")
    num_cores = 2
    kfull = dim // _SUB
    nbg = batch // _SUB

    @functools.partial(
        pl.kernel,
        mesh=mesh,
        out_type=jax.ShapeDtypeStruct((dpad // _RPW, _RPW * batch), jnp.float32),
        scratch_types=[
            pltpu.VMEM((_RPW // _SUB, _SUB, _SUB, 128), jnp.float32),
            pltpu.VMEM((_RPW * batch,), jnp.float32),
            pltpu.SemaphoreType.DMA,
        ],
        compiler_params=pltpu.CompilerParams(
            use_tc_tiling_on_sc=True, needs_layout_passes=False
        ),
    )
    def gather_kernel(xt_hbm, out_hbm, buf_v, diag_v, sem):
        wid = lax.axis_index("s") * num_cores + lax.axis_index("c")
        q = lax.shift_right_logical(wid, 1)
        for p in range(2):

            @pl.when(lax.bitwise_and(wid, 1) == p)
            def _branch(p=p):
                for u in range(_RPW // _SUB):
                    r0 = _RPW * wid + _SUB * u
                    live = _SUB * wid + u < kfull
                    for g in range(nbg):

                        @pl.when(live)
                        def _issue(u=u, g=g, r0=r0, p=p):
                            band = xt_hbm.at[
                                pl.ds(r0, _SUB),
                                pl.ds(_SUB * g, _SUB),
                                pl.ds(128 * q, 128),
                            ]
                            pltpu.async_copy(
                                band.at[:, :, pl.ds(_SUB * (_SUB * p + u), _SUB)],
                                buf_v.at[u, :, :, pl.ds(_SUB * g, _SUB)],
                                sem,
                            )

        for u in range(_RPW // _SUB):
            live = _SUB * wid + u < kfull
            for g in range(nbg):

                @pl.when(live)
                def _drain(u=u, g=g):
                    pltpu.make_async_copy(
                        xt_hbm.at[pl.ds(0, _SUB), pl.ds(0, _SUB), pl.ds(0, _SUB)],
                        buf_v.at[u, :, :, pl.ds(_SUB * g, _SUB)],
                        sem,
                    ).wait()

        lane = lax.iota(jnp.int32, 16)
        bg = lax.shift_right_logical(lane, 3)
        bsub = lax.bitwise_and(lane, 7)
        for u in range(_RPW // _SUB):
            for l in range(_SUB):
                for half in range(nbg // 2):
                    vals = plsc.load_gather(
                        buf_v,
                        [lane * 0 + u, lane * 0 + l, bsub, lane - bsub + 16 * half + l],
                    )
                    diag_v[pl.ds((_SUB * u + l) * batch + 16 * half, 16)] = vals
        pltpu.sync_copy(diag_v, out_hbm.at[wid])

    return gather_kernel(xt)


def _project_tc(diag_t, tail, p_pad, batch, dim, dpad, s):
    rem = dim % _SUB
    base = dim - rem

    def body(d_ref, t_ref, p_ref, o_ref):
        d = d_ref[...]
        row = lax.broadcasted_iota(jnp.int32, (dpad, batch), 0)
        d = jnp.where(row >= base, 0.0, d)
        for x in range(rem):
            fix = t_ref[:, x * rem + x][None, :]
            d = jnp.where(row == base + x, fix, d)
        o_ref[...] = jax.lax.dot_general(
            d,
            p_ref[...],
            (((0,), (0,)), ((), ())),
            preferred_element_type=jnp.float32,
        )

    return pl.pallas_call(
        body,
        out_shape=jax.ShapeDtypeStruct((batch, s), jnp.float32),
    )(diag_t, tail, p_pad)


def kernel(input, P):
    batch, dim, _ = input.shape
    s = P.shape[1]
    dpad = ((dim + 127) // 128) * 128
    rem = dim % _SUB
    base = dim - rem

    xt = jnp.transpose(input, (1, 0, 2))
    diag3 = _diag_gather_sc(xt, batch, dim, dpad)
    diag_t = diag3.reshape(dpad, batch)
    p_pad = jnp.pad(P, ((0, dpad - dim), (0, 0)))
    tail = input[:, base:, base:].reshape(batch, max(rem * rem, 1))
    return _project_tc(diag_t, tail, p_pad, batch, dim, dpad, s)


# speedup vs baseline: 2.1518x; 2.1518x over previous
import jax
import jax.numpy as jnp
from jax import lax
from jax.experimental import pallas as pl


def _diag_project_body(dim, x_ref, p_ref, o_ref):
    t = pl.program_id(0)
    blk = x_ref[...]
    rr = lax.broadcasted_iota(jnp.int32, (128, 1, 128), 0)
    cc = lax.broadcasted_iota(jnp.int32, (128, 1, 128), 2)
    z = jnp.where(rr == cc, blk, 0.0)
    g = jnp.sum(z, axis=0)
    lane = lax.broadcasted_iota(jnp.int32, g.shape, 1)
    g = jnp.where(128 * t + lane < dim, g, 0.0)
    contrib = jnp.dot(g, p_ref[...], preferred_element_type=jnp.float32)

    @pl.when(t == 0)
    def _init():
        o_ref[...] = contrib

    @pl.when(t != 0)
    def _acc():
        o_ref[...] += contrib


def kernel(input, P):
    batch, dim, _ = input.shape
    s = P.shape[1]
    dpad = ((dim + 127) // 128) * 128
    nblk = dpad // 128

    xt = jnp.transpose(input, (1, 0, 2))
    p_pad = jnp.pad(P, ((0, dpad - dim), (0, 0)))
    return pl.pallas_call(
        lambda x, p, o: _diag_project_body(dim, x, p, o),
        grid=(nblk,),
        in_specs=[
            pl.BlockSpec((128, batch, 128), lambda t: (t, 0, t)),
            pl.BlockSpec((128, s), lambda t: (t, 0)),
        ],
        out_specs=pl.BlockSpec((batch, s), lambda t: (0, 0)),
        out_shape=jax.ShapeDtypeStruct((batch, s), jnp.float32),
    )(xt, p_pad)
